# baseline (speedup 1.0000x reference)
import functools

import jax
import jax.numpy as jnp
from jax import lax
from jax.experimental import pallas as pl
from jax.experimental.pallas import tpu as pltpu
from jax.experimental.pallas import tpu_sc as plsc

B = 16384
L = 1200
VOCAB = 10
EMB = 4

NC = 2
NS = 16
NW = NC * NS
LANES = 16

LPAD = 1280
LP = LPAD // 4
NSL = LP // LANES
NSPLIT = 4
CHUNK_ROWS = 32
CHUNK_WORDS = CHUNK_ROWS * LP
RB = 8

_GATHER_DNUMS = lax.GatherDimensionNumbers(
    offset_dims=(), collapsed_slice_dims=(0,), start_index_map=(0,))


def _take16(vec, idx):
    return lax.gather(
        vec, idx[:, None], dimension_numbers=_GATHER_DNUMS, slice_sizes=(1,),
        mode=lax.GatherScatterMode.PROMISE_IN_BOUNDS)


def _sc_body(rows_per_w, nchunk,
             a_hbm, tblT_hbm, params_hbm, w2_hbm, out_hbm,
             buf0, buf1, w2_v, tbl_v, par_v, out_v, sem0, sem1):
    wid = lax.axis_index("s") * NC + lax.axis_index("c")
    base_row = wid * rows_per_w

    pltpu.sync_copy(tblT_hbm, tbl_v)
    pltpu.sync_copy(params_hbm, par_v)
    pltpu.sync_copy(w2_hbm, w2_v)

    par = par_v[...]
    t_vec = par[EMB] + jnp.zeros((LANES,), jnp.float32)
    for e in range(EMB):
        t_vec = t_vec + tbl_v[e, :] * par[e]
    b2s = par[EMB + 1]
    lane_iota = lax.iota(jnp.int32, LANES)

    def hsum(v):
        for sh in (8, 4, 2, 1):
            v = v + _take16(v, lane_iota ^ sh)
        return v

    def chunk_src(c):
        return a_hbm.at[pl.ds((base_row + c * CHUNK_ROWS) * LP, CHUNK_WORDS)]

    pltpu.async_copy(chunk_src(0), buf0, sem0)

    def compute(buf, c):
        for g in range(CHUNK_ROWS // LANES):
            gvec = jnp.zeros((LANES,), jnp.float32)
            for h in range(LANES // RB):
                def lbody(ks, accs):
                    k0 = ks * LANES
                    w2v = tuple(w2_v[pl.ds(j * LP + k0, LANES)]
                                for j in range(4))
                    out = []
                    for r in range(RB):
                        row = g * LANES + h * RB + r
                        w = buf[pl.ds(row * LP + k0, LANES)]
                        acc = accs[r]
                        acc = acc + _take16(t_vec, w & 15) * w2v[0]
                        acc = acc + _take16(t_vec, (w >> 8) & 15) * w2v[1]
                        acc = acc + _take16(t_vec, (w >> 16) & 15) * w2v[2]
                        acc = acc + _take16(
                            t_vec, lax.shift_right_logical(w, 24)) * w2v[3]
                        out.append(acc)
                    return tuple(out)

                accs = lax.fori_loop(
                    0, NSL, lbody,
                    tuple(jnp.zeros((LANES,), jnp.float32)
                          for _ in range(RB)))
                for r in range(RB):
                    sv = hsum(accs[r]) + b2s
                    gvec = jnp.where(lane_iota == h * RB + r, sv, gvec)
            out_v[pl.ds(c * CHUNK_ROWS + g * LANES, LANES)] = gvec

    def loop_body(c, _):
        nxt = c + 1

        @pl.when((c & 1) == 0)
        def _even():
            @pl.when(nxt < nchunk)
            def _():
                pltpu.async_copy(chunk_src(nxt), buf1, sem1)
            pltpu.make_async_copy(chunk_src(0), buf0, sem0).wait()
            compute(buf0, c)

        @pl.when((c & 1) == 1)
        def _odd():
            @pl.when(nxt < nchunk)
            def _():
                pltpu.async_copy(chunk_src(nxt), buf0, sem0)
            pltpu.make_async_copy(chunk_src(0), buf1, sem1).wait()
            compute(buf1, c)

        return 0

    lax.fori_loop(0, nchunk, loop_body, 0)

    pltpu.sync_copy(out_v, out_hbm.at[pl.ds(base_row, rows_per_w)])


def _make_run(rows):
    rows_per_w = rows // NW
    nchunk = rows_per_w // CHUNK_ROWS
    mesh = plsc.VectorSubcoreMesh(core_axis_name="c", subcore_axis_name="s")
    return pl.kernel(
        functools.partial(_sc_body, rows_per_w, nchunk),
        mesh=mesh,
        out_type=jax.ShapeDtypeStruct((rows,), jnp.float32),
        scratch_types=[
            pltpu.VMEM((CHUNK_WORDS,), jnp.int32),
            pltpu.VMEM((CHUNK_WORDS,), jnp.int32),
            pltpu.VMEM((4 * LP,), jnp.float32),
            pltpu.VMEM((EMB, LANES), jnp.float32),
            pltpu.VMEM((LANES,), jnp.float32),
            pltpu.VMEM((rows_per_w,), jnp.float32),
            pltpu.SemaphoreType.DMA,
            pltpu.SemaphoreType.DMA,
        ],
    )


@jax.jit
def kernel(a_input, table, w1, b1, w2, b2):
    tblT = jnp.pad(table.T, ((0, 0), (0, LANES - VOCAB)))
    params = jnp.concatenate(
        [w1.reshape(EMB), b1.reshape(1), b2.reshape(1),
         jnp.zeros((LANES - EMB - 2,), jnp.float32)])
    w2r = jnp.pad(w2.reshape(L), (0, LPAD - L)).reshape(LP, 4).T.reshape(-1)

    rows = B // NSPLIT
    run = _make_run(rows)
    outs = []
    for k in range(NSPLIT):
        a3 = jnp.pad(lax.slice_in_dim(a_input, k * rows, (k + 1) * rows),
                     ((0, 0), (0, LPAD - L))).reshape(rows, LP, 4)
        ap = (a3[:, :, 0] + a3[:, :, 1] * 256 + a3[:, :, 2] * 65536
              + a3[:, :, 3] * 16777216).reshape(rows * LP)
        outs.append(run(ap, tblT, params, w2r))
    out = outs[0] if NSPLIT == 1 else jnp.concatenate(outs)
    return out.reshape(B, 1)


# speedup vs baseline: 4.8629x; 4.8629x over previous
import jax
import jax.numpy as jnp
from jax import lax
from jax.experimental import pallas as pl
from jax.experimental.pallas import tpu as pltpu
from jax.experimental.pallas import tpu_sc as plsc

B = 16384
L = 1200
VOCAB = 10
EMB = 4

NC = 2
NS = 16
NW = NC * NS
LANES = 16

ROWS_PER_W = B // NW
CHUNK_ROWS = 32
NCHUNK = ROWS_PER_W // CHUNK_ROWS
RB = 8
NSLICE = L // LANES

_GATHER_DNUMS = lax.GatherDimensionNumbers(
    offset_dims=(), collapsed_slice_dims=(0,), start_index_map=(0,))


def _take16(vec, idx):
    return lax.gather(
        vec, idx[:, None], dimension_numbers=_GATHER_DNUMS, slice_sizes=(1,),
        mode=lax.GatherScatterMode.PROMISE_IN_BOUNDS)


def _sc_kernel(a_hbm, tblT_hbm, params_hbm, w2_hbm, out_hbm,
               buf0, buf1, w2_v, tbl_v, par_v, out_v, sem0, sem1):
    wid = lax.axis_index("s") * NC + lax.axis_index("c")
    base_row = wid * ROWS_PER_W

    pltpu.sync_copy(tblT_hbm, tbl_v)
    pltpu.sync_copy(params_hbm, par_v)
    pltpu.sync_copy(w2_hbm, w2_v)

    par = par_v[...]
    t_vec = par[EMB] + jnp.zeros((LANES,), jnp.float32)
    for e in range(EMB):
        t_vec = t_vec + tbl_v[e, :] * par[e]
    b2s = par[EMB + 1]
    lane_iota = lax.iota(jnp.int32, LANES)

    def hsum(v):
        for sh in (8, 4, 2, 1):
            v = v + _take16(v, lane_iota ^ sh)
        return v

    def chunk_src(c):
        return a_hbm.at[pl.ds(base_row + c * CHUNK_ROWS, CHUNK_ROWS), :]

    pltpu.async_copy(chunk_src(0), buf0, sem0)

    def compute(buf, c):
        for g in range(CHUNK_ROWS // LANES):
            gvec = jnp.zeros((LANES,), jnp.float32)
            for h in range(LANES // RB):
                def lbody(ls, accs):
                    off = ls * LANES
                    w2s = w2_v[pl.ds(off, LANES)]
                    out = []
                    for r in range(RB):
                        idx = buf[g * LANES + h * RB + r, pl.ds(off, LANES)]
                        val = _take16(t_vec, idx)
                        out.append(accs[r] + val * w2s)
                    return tuple(out)

                accs = lax.fori_loop(
                    0, NSLICE, lbody,
                    tuple(jnp.zeros((LANES,), jnp.float32)
                          for _ in range(RB)))
                for r in range(RB):
                    sv = hsum(accs[r]) + b2s
                    gvec = jnp.where(lane_iota == h * RB + r, sv, gvec)
            out_v[pl.ds(c * CHUNK_ROWS + g * LANES, LANES)] = gvec

    def loop_body(c, _):
        nxt = c + 1

        @pl.when((c & 1) == 0)
        def _even():
            @pl.when(nxt < NCHUNK)
            def _():
                pltpu.async_copy(chunk_src(nxt), buf1, sem1)
            pltpu.make_async_copy(chunk_src(0), buf0, sem0).wait()
            compute(buf0, c)

        @pl.when((c & 1) == 1)
        def _odd():
            @pl.when(nxt < NCHUNK)
            def _():
                pltpu.async_copy(chunk_src(nxt), buf0, sem0)
            pltpu.make_async_copy(chunk_src(0), buf1, sem1).wait()
            compute(buf1, c)

        return 0

    lax.fori_loop(0, NCHUNK, loop_body, 0)

    pltpu.sync_copy(out_v, out_hbm.at[pl.ds(base_row, ROWS_PER_W)])


@jax.jit
def kernel(a_input, table, w1, b1, w2, b2):
    tblT = jnp.pad(table.T, ((0, 0), (0, LANES - VOCAB)))
    params = jnp.concatenate(
        [w1.reshape(EMB), b1.reshape(1), b2.reshape(1),
         jnp.zeros((LANES - EMB - 2,), jnp.float32)])
    w2f = w2.reshape(L)

    mesh = plsc.VectorSubcoreMesh(core_axis_name="c", subcore_axis_name="s")
    run = pl.kernel(
        _sc_kernel,
        mesh=mesh,
        out_type=jax.ShapeDtypeStruct((B,), jnp.float32),
        scratch_types=[
            pltpu.VMEM((CHUNK_ROWS, L), jnp.int32),
            pltpu.VMEM((CHUNK_ROWS, L), jnp.int32),
            pltpu.VMEM((L,), jnp.float32),
            pltpu.VMEM((EMB, LANES), jnp.float32),
            pltpu.VMEM((LANES,), jnp.float32),
            pltpu.VMEM((ROWS_PER_W,), jnp.float32),
            pltpu.SemaphoreType.DMA,
            pltpu.SemaphoreType.DMA,
        ],
    )
    return run(a_input, tblT, params, w2f).reshape(B, 1)


# speedup vs baseline: 4.8895x; 1.0055x over previous
import jax
import jax.numpy as jnp
from jax import lax
from jax.experimental import pallas as pl
from jax.experimental.pallas import tpu as pltpu
from jax.experimental.pallas import tpu_sc as plsc

B = 16384
L = 1200
VOCAB = 10
EMB = 4

NC = 2
NS = 16
NW = NC * NS
LANES = 16

ROWS_PER_W = B // NW
CHUNK_ROWS = 32
NCHUNK = ROWS_PER_W // CHUNK_ROWS
RB = 16
NSLICE = L // LANES

_GATHER_DNUMS = lax.GatherDimensionNumbers(
    offset_dims=(), collapsed_slice_dims=(0,), start_index_map=(0,))


def _take16(vec, idx):
    return lax.gather(
        vec, idx[:, None], dimension_numbers=_GATHER_DNUMS, slice_sizes=(1,),
        mode=lax.GatherScatterMode.PROMISE_IN_BOUNDS)


def _sc_kernel(a_hbm, tblT_hbm, params_hbm, w2_hbm, out_hbm,
               buf0, buf1, w2_v, tbl_v, par_v, out_v, sem0, sem1):
    wid = lax.axis_index("s") * NC + lax.axis_index("c")
    base_row = wid * ROWS_PER_W

    pltpu.sync_copy(tblT_hbm, tbl_v)
    pltpu.sync_copy(params_hbm, par_v)
    pltpu.sync_copy(w2_hbm, w2_v)

    par = par_v[...]
    t_vec = par[EMB] + jnp.zeros((LANES,), jnp.float32)
    for e in range(EMB):
        t_vec = t_vec + tbl_v[e, :] * par[e]
    b2s = par[EMB + 1]
    lane_iota = lax.iota(jnp.int32, LANES)

    def hsum(v):
        for sh in (8, 4, 2, 1):
            v = v + _take16(v, lane_iota ^ sh)
        return v

    def chunk_src(c):
        return a_hbm.at[pl.ds(base_row + c * CHUNK_ROWS, CHUNK_ROWS), :]

    pltpu.async_copy(chunk_src(0), buf0, sem0)

    def compute(buf, c):
        for g in range(CHUNK_ROWS // LANES):
            gvec = jnp.zeros((LANES,), jnp.float32)
            for h in range(LANES // RB):
                def lbody(ls, accs):
                    off = ls * LANES
                    w2s = w2_v[pl.ds(off, LANES)]
                    out = []
                    for r in range(RB):
                        idx = buf[g * LANES + h * RB + r, pl.ds(off, LANES)]
                        val = _take16(t_vec, idx)
                        out.append(accs[r] + val * w2s)
                    return tuple(out)

                accs = lax.fori_loop(
                    0, NSLICE, lbody,
                    tuple(jnp.zeros((LANES,), jnp.float32)
                          for _ in range(RB)))
                for r in range(RB):
                    sv = hsum(accs[r]) + b2s
                    gvec = jnp.where(lane_iota == h * RB + r, sv, gvec)
            out_v[pl.ds(c * CHUNK_ROWS + g * LANES, LANES)] = gvec

    def loop_body(c, _):
        nxt = c + 1

        @pl.when((c & 1) == 0)
        def _even():
            @pl.when(nxt < NCHUNK)
            def _():
                pltpu.async_copy(chunk_src(nxt), buf1, sem1)
            pltpu.make_async_copy(chunk_src(0), buf0, sem0).wait()
            compute(buf0, c)

        @pl.when((c & 1) == 1)
        def _odd():
            @pl.when(nxt < NCHUNK)
            def _():
                pltpu.async_copy(chunk_src(nxt), buf0, sem0)
            pltpu.make_async_copy(chunk_src(0), buf1, sem1).wait()
            compute(buf1, c)

        return 0

    lax.fori_loop(0, NCHUNK, loop_body, 0)

    pltpu.sync_copy(out_v, out_hbm.at[pl.ds(base_row, ROWS_PER_W)])


@jax.jit
def kernel(a_input, table, w1, b1, w2, b2):
    tblT = jnp.pad(table.T, ((0, 0), (0, LANES - VOCAB)))
    params = jnp.concatenate(
        [w1.reshape(EMB), b1.reshape(1), b2.reshape(1),
         jnp.zeros((LANES - EMB - 2,), jnp.float32)])
    w2f = w2.reshape(L)

    mesh = plsc.VectorSubcoreMesh(core_axis_name="c", subcore_axis_name="s")
    run = pl.kernel(
        _sc_kernel,
        mesh=mesh,
        out_type=jax.ShapeDtypeStruct((B,), jnp.float32),
        scratch_types=[
            pltpu.VMEM((CHUNK_ROWS, L), jnp.int32),
            pltpu.VMEM((CHUNK_ROWS, L), jnp.int32),
            pltpu.VMEM((L,), jnp.float32),
            pltpu.VMEM((EMB, LANES), jnp.float32),
            pltpu.VMEM((LANES,), jnp.float32),
            pltpu.VMEM((ROWS_PER_W,), jnp.float32),
            pltpu.SemaphoreType.DMA,
            pltpu.SemaphoreType.DMA,
        ],
    )
    return run(a_input, tblT, params, w2f).reshape(B, 1)


# speedup vs baseline: 4.8947x; 1.0011x over previous
import jax
import jax.numpy as jnp
from jax import lax
from jax.experimental import pallas as pl
from jax.experimental.pallas import tpu as pltpu
from jax.experimental.pallas import tpu_sc as plsc

B = 16384
L = 1200
VOCAB = 10
EMB = 4

NC = 2
NS = 16
NW = NC * NS
LANES = 16

ROWS_PER_W = B // NW
CHUNK_ROWS = 32
NCHUNK = ROWS_PER_W // CHUNK_ROWS
RB = 16
NSLICE = L // LANES

_GATHER_DNUMS = lax.GatherDimensionNumbers(
    offset_dims=(), collapsed_slice_dims=(0,), start_index_map=(0,))


def _take16(vec, idx):
    return lax.gather(
        vec, idx[:, None], dimension_numbers=_GATHER_DNUMS, slice_sizes=(1,),
        mode=lax.GatherScatterMode.PROMISE_IN_BOUNDS)


def _sc_kernel(a_hbm, tblT_hbm, params_hbm, w2_hbm, out_hbm,
               buf0, buf1, w2_v, tbl_v, par_v, out_v, sem0, sem1):
    wid = lax.axis_index("s") * NC + lax.axis_index("c")
    base_row = wid * ROWS_PER_W

    pltpu.sync_copy(tblT_hbm, tbl_v)
    pltpu.sync_copy(params_hbm, par_v)
    pltpu.sync_copy(w2_hbm, w2_v)

    par = par_v[...]
    t_vec = par[EMB] + jnp.zeros((LANES,), jnp.float32)
    for e in range(EMB):
        t_vec = t_vec + tbl_v[e, :] * par[e]
    b2s = par[EMB + 1]
    lane_iota = lax.iota(jnp.int32, LANES)

    def hsum(v):
        for sh in (8, 4, 2, 1):
            v = v + _take16(v, lane_iota ^ sh)
        return v

    def chunk_src(c):
        return a_hbm.at[pl.ds(base_row + c * CHUNK_ROWS, CHUNK_ROWS), :]

    pltpu.async_copy(chunk_src(0), buf0, sem0)

    def compute(buf, c):
        for g in range(CHUNK_ROWS // LANES):
            gvec = jnp.zeros((LANES,), jnp.float32)
            for h in range(LANES // RB):
                def lbody(ls, accs):
                    off = ls * LANES
                    w2s = w2_v[pl.ds(off, LANES)]
                    out = []
                    for r in range(RB):
                        idx = buf[g * LANES + h * RB + r, pl.ds(off, LANES)]
                        val = _take16(t_vec, idx)
                        out.append(accs[r] + val * w2s)
                    return tuple(out)

                accs = lax.fori_loop(
                    0, NSLICE, lbody,
                    tuple(jnp.zeros((LANES,), jnp.float32)
                          for _ in range(RB)))
                for r in range(RB):
                    sv = hsum(accs[r]) + b2s
                    gvec = jnp.where(lane_iota == h * RB + r, sv, gvec)
            out_v[pl.ds(c * CHUNK_ROWS + g * LANES, LANES)] = gvec

    def loop_body(c, _):
        nxt = c + 1

        @pl.when((c & 1) == 0)
        def _even():
            @pl.when(nxt < NCHUNK)
            def _():
                pltpu.async_copy(chunk_src(nxt), buf1, sem1)
            pltpu.make_async_copy(chunk_src(0), buf0, sem0).wait()
            compute(buf0, c)

        @pl.when((c & 1) == 1)
        def _odd():
            @pl.when(nxt < NCHUNK)
            def _():
                pltpu.async_copy(chunk_src(nxt), buf0, sem0)
            pltpu.make_async_copy(chunk_src(0), buf1, sem1).wait()
            compute(buf1, c)

        return 0

    lax.fori_loop(0, NCHUNK, loop_body, 0)

    pltpu.sync_copy(out_v, out_hbm.at[pl.ds(base_row, ROWS_PER_W)])


@jax.jit
def kernel(a_input, table, w1, b1, w2, b2):
    tblT = jnp.pad(table.T, ((0, 0), (0, LANES - VOCAB)))
    params = jnp.concatenate(
        [w1.reshape(EMB), b1.reshape(1), b2.reshape(1),
         jnp.zeros((LANES - EMB - 2,), jnp.float32)])
    w2f = w2.reshape(L)

    mesh = plsc.VectorSubcoreMesh(core_axis_name="c", subcore_axis_name="s")
    run = pl.kernel(
        _sc_kernel,
        mesh=mesh,
        compiler_params=pltpu.CompilerParams(
            allow_input_fusion=[True, True, True, True]),
        out_type=jax.ShapeDtypeStruct((B,), jnp.float32),
        scratch_types=[
            pltpu.VMEM((CHUNK_ROWS, L), jnp.int32),
            pltpu.VMEM((CHUNK_ROWS, L), jnp.int32),
            pltpu.VMEM((L,), jnp.float32),
            pltpu.VMEM((EMB, LANES), jnp.float32),
            pltpu.VMEM((LANES,), jnp.float32),
            pltpu.VMEM((ROWS_PER_W,), jnp.float32),
            pltpu.SemaphoreType.DMA,
            pltpu.SemaphoreType.DMA,
        ],
    )
    return run(a_input, tblT, params, w2f).reshape(B, 1)


# speedup vs baseline: 4.8998x; 1.0010x over previous
import jax
import jax.numpy as jnp
from jax import lax
from jax.experimental import pallas as pl
from jax.experimental.pallas import tpu as pltpu
from jax.experimental.pallas import tpu_sc as plsc

B = 16384
L = 1200
VOCAB = 10
EMB = 4

NC = 2
NS = 16
NW = NC * NS
LANES = 16

ROWS_PER_W = B // NW
CHUNK_ROWS = 32
NCHUNK = ROWS_PER_W // CHUNK_ROWS
RB = 16
NSLICE = L // LANES

_GATHER_DNUMS = lax.GatherDimensionNumbers(
    offset_dims=(), collapsed_slice_dims=(0,), start_index_map=(0,))


def _take16(vec, idx):
    return lax.gather(
        vec, idx[:, None], dimension_numbers=_GATHER_DNUMS, slice_sizes=(1,),
        mode=lax.GatherScatterMode.PROMISE_IN_BOUNDS)


def _sc_kernel(a_hbm, tblT_hbm, params_hbm, w2_hbm, out_hbm,
               buf0, buf1, w2_v, tbl_v, par_v, out_v, sem0, sem1):
    wid = lax.axis_index("s") * NC + lax.axis_index("c")
    base_row = wid * ROWS_PER_W

    pltpu.sync_copy(tblT_hbm, tbl_v)
    pltpu.sync_copy(params_hbm, par_v)
    pltpu.sync_copy(w2_hbm, w2_v)

    par = par_v[...]
    t_vec = par[EMB] + jnp.zeros((LANES,), jnp.float32)
    for e in range(EMB):
        t_vec = t_vec + tbl_v[e, :] * par[e]
    b2s = par[EMB + 1]
    lane_iota = lax.iota(jnp.int32, LANES)

    def hsum(v):
        for sh in (8, 4, 2, 1):
            v = v + _take16(v, lane_iota ^ sh)
        return v

    def chunk_src(c):
        return a_hbm.at[pl.ds(base_row + c * CHUNK_ROWS, CHUNK_ROWS), :]

    pltpu.async_copy(chunk_src(0), buf0, sem0)

    def compute(buf, c):
        for g in range(CHUNK_ROWS // LANES):
            gvec = jnp.zeros((LANES,), jnp.float32)
            for h in range(LANES // RB):
                def lbody(ls, accs):
                    off = ls * LANES
                    w2s = w2_v[pl.ds(off, LANES)]
                    out = []
                    for r in range(RB):
                        idx = buf[g * LANES + h * RB + r, pl.ds(off, LANES)]
                        val = _take16(t_vec, idx)
                        out.append(accs[r] + val * w2s)
                    return tuple(out)

                accs = lax.fori_loop(
                    0, NSLICE, lbody,
                    tuple(jnp.zeros((LANES,), jnp.float32)
                          for _ in range(RB)))
                for r in range(RB):
                    sv = hsum(accs[r]) + b2s
                    gvec = jnp.where(lane_iota == h * RB + r, sv, gvec)
            out_v[pl.ds(c * CHUNK_ROWS + g * LANES, LANES)] = gvec

    def loop_body(c, _):
        nxt = c + 1

        @pl.when((c & 1) == 0)
        def _even():
            @pl.when(nxt < NCHUNK)
            def _():
                pltpu.async_copy(chunk_src(nxt), buf1, sem1)
            pltpu.make_async_copy(chunk_src(0), buf0, sem0).wait()
            compute(buf0, c)

        @pl.when((c & 1) == 1)
        def _odd():
            @pl.when(nxt < NCHUNK)
            def _():
                pltpu.async_copy(chunk_src(nxt), buf0, sem0)
            pltpu.make_async_copy(chunk_src(0), buf1, sem1).wait()
            compute(buf1, c)

        return 0

    lax.fori_loop(0, NCHUNK, loop_body, 0)

    pltpu.sync_copy(out_v, out_hbm.at[pl.ds(base_row, ROWS_PER_W)])


@jax.jit
def kernel(a_input, table, w1, b1, w2, b2):
    tblT = jnp.pad(table.T, ((0, 0), (0, LANES - VOCAB)))
    params = jnp.concatenate(
        [w1.reshape(EMB), b1.reshape(1), b2.reshape(1),
         jnp.zeros((LANES - EMB - 2,), jnp.float32)])
    w2f = w2.reshape(L)

    mesh = plsc.VectorSubcoreMesh(core_axis_name="c", subcore_axis_name="s")
    run = pl.kernel(
        _sc_kernel,
        mesh=mesh,
        out_type=jax.ShapeDtypeStruct((B,), jnp.float32),
        scratch_types=[
            pltpu.VMEM((CHUNK_ROWS, L), jnp.int32),
            pltpu.VMEM((CHUNK_ROWS, L), jnp.int32),
            pltpu.VMEM((L,), jnp.float32),
            pltpu.VMEM((EMB, LANES), jnp.float32),
            pltpu.VMEM((LANES,), jnp.float32),
            pltpu.VMEM((ROWS_PER_W,), jnp.float32),
            pltpu.SemaphoreType.DMA,
            pltpu.SemaphoreType.DMA,
        ],
    )
    return run(a_input, tblT, params, w2f).reshape(B, 1)
